# baseline (speedup 1.0000x reference)
import jax
import jax.numpy as jnp
from jax import lax
from jax.experimental import pallas as pl
from jax.experimental.pallas import tpu as pltpu
from jax.experimental.pallas import tpu_sc as plsc

N_NODES = 10000
N_CLASSES = 40
K_STEPS = 10
ALPHA = 0.1

NC, NS = 2, 16
NW = NC * NS
NP = 10240
PAD_NODE = N_NODES
DP = 48
EP = 327680
E_CHUNK = 128
N_CHUNKS_PER_TILE = EP // NW // E_CHUNK
ROWS_PER_TILE = NP // NS
QROWS = ROWS_PER_TILE // 8

_mesh = lambda: plsc.VectorSubcoreMesh(
    core_axis_name="c", subcore_axis_name="s", num_cores=NC, num_subcores=NS)
_SC_PARAMS = pltpu.CompilerParams(use_tc_tiling_on_sc=False)


def _mlp_body(x_ref, w1_ref, b1_ref, w2_ref, b2_ref, p0_ref, p1_ref,
              u0_ref, c_ref, g_ref):
    x = x_ref[...]
    h = jax.nn.relu(jnp.dot(x, w1_ref[...], preferred_element_type=jnp.float32)
                    + b1_ref[...][None, :])
    h = jnp.dot(h, w2_ref[...], preferred_element_type=jnp.float32) \
        + b2_ref[...][None, :]
    deg = p0_ref[...][:, 0:1] + p1_ref[...][:, 0:1] + 1.0
    dinv = lax.rsqrt(deg)
    row = lax.broadcasted_iota(jnp.int32, (NP, 1), 0)
    live = row < N_NODES
    u0 = jnp.where(live, dinv * h, 0.0)
    u0_ref[...] = u0
    c_ref[...] = jnp.where(live, jnp.ones((NP, DP), jnp.float32) / deg, 0.0)
    g_ref[...] = ALPHA * u0


def _mlp_call(x_pad, w1, b1, w2p, b2p, p0, p1):
    return pl.pallas_call(
        _mlp_body,
        out_shape=(
            jax.ShapeDtypeStruct((NP, DP), jnp.float32),
            jax.ShapeDtypeStruct((NP, DP), jnp.float32),
            jax.ShapeDtypeStruct((NP, DP), jnp.float32),
        ),
    )(x_pad, w1, b1, w2p, b2p, p0, p1)


def _out_body(pp_ref, u_ref, c_ref, g_ref, p0_ref, p1_ref, o_ref):
    u10 = (1.0 - ALPHA) * c_ref[...] * (
        pp_ref[0] + pp_ref[1] + u_ref[...]) + g_ref[...]
    deg = p0_ref[...][:N_NODES, 0:1] + p1_ref[...][:N_NODES, 0:1] + 1.0
    z = jnp.sqrt(deg) * u10[:N_NODES, :N_CLASSES]
    m = jnp.max(z, axis=1, keepdims=True)
    e = jnp.exp(z - m)
    o_ref[...] = (z - m) - jnp.log(jnp.sum(e, axis=1, keepdims=True))


def _out_call(parts, u, cb, g, p0, p1):
    return pl.pallas_call(
        _out_body,
        out_shape=jax.ShapeDtypeStruct((N_NODES, N_CLASSES), jnp.float32),
    )(parts, u, cb, g, p0, p1)


def _deg_body(dst_hbm, ones_hbm, zeros_hbm, out_hbm, dstv, onesv, acc, *sems):
    c = lax.axis_index("c")
    s = lax.axis_index("s")
    wid = c * NS + s
    pltpu.sync_copy(zeros_hbm.at[pl.ds(s * ROWS_PER_TILE, ROWS_PER_TILE)],
                    acc.at[pl.ds(s * ROWS_PER_TILE, ROWS_PER_TILE)])
    pltpu.sync_copy(ones_hbm, onesv)
    pltpu.sync_copy(dst_hbm.at[pl.ds(wid * N_CHUNKS_PER_TILE, N_CHUNKS_PER_TILE)],
                    dstv)
    plsc.subcore_barrier()

    def body(t, _):
        for b in range(NSLOT):
            j = t * NSLOT + b

            @pl.when(t > 0)
            def _():
                pltpu.make_async_copy(onesv, acc.at[dstv.at[j]],
                                      sems[b]).wait()
            pltpu.async_copy(onesv, acc.at[dstv.at[j]], sems[b], add=True)
        return 0

    lax.fori_loop(0, N_CHUNKS_PER_TILE // NSLOT, body, 0)
    for b in range(NSLOT):
        j = N_CHUNKS_PER_TILE - NSLOT + b
        pltpu.make_async_copy(onesv, acc.at[dstv.at[j]], sems[b]).wait()
    plsc.subcore_barrier()
    pltpu.sync_copy(acc.at[pl.ds(s * ROWS_PER_TILE, ROWS_PER_TILE)],
                    out_hbm.at[c, pl.ds(s * ROWS_PER_TILE, ROWS_PER_TILE)])


def _deg_call(dst2d, ones_h, zeros16_h):
    return pl.kernel(
        _deg_body,
        out_type=jax.ShapeDtypeStruct((NC, NP, 16), jnp.float32),
        mesh=_mesh(),
        compiler_params=_SC_PARAMS,
        scratch_types=[
            pltpu.VMEM((N_CHUNKS_PER_TILE, E_CHUNK), jnp.int32),
            pltpu.VMEM((E_CHUNK, 16), jnp.float32),
            pltpu.VMEM_SHARED((NP, 16), jnp.float32),
        ] + [pltpu.SemaphoreType.DMA for _ in range(NSLOT)],
    )(dst2d, ones_h, zeros16_h)


NSLOT = 8


def _edge_phase(src_hbm, dst_hbm, u_src, zeros_hbm, out_hbm,
                srcv, dstv, rows, acc, gsems, ssems, s, c, wid):
    pltpu.sync_copy(zeros_hbm.at[pl.ds(s * ROWS_PER_TILE, ROWS_PER_TILE)],
                    acc.at[pl.ds(s * ROWS_PER_TILE, ROWS_PER_TILE)])
    pltpu.sync_copy(src_hbm.at[pl.ds(wid * N_CHUNKS_PER_TILE, N_CHUNKS_PER_TILE)],
                    srcv)
    pltpu.sync_copy(dst_hbm.at[pl.ds(wid * N_CHUNKS_PER_TILE, N_CHUNKS_PER_TILE)],
                    dstv)
    plsc.subcore_barrier()

    for b in range(NSLOT):
        pltpu.async_copy(u_src.at[srcv.at[b]], rows[b], gsems[b])

    def body(t, _):
        for b in range(NSLOT):
            j = t * NSLOT + b
            pltpu.make_async_copy(u_src.at[srcv.at[j]], rows[b], gsems[b]).wait()
            pltpu.async_copy(rows[b], acc.at[dstv.at[j]], ssems[b], add=True)
        for b in range(NSLOT):
            j = t * NSLOT + b
            jn = j + NSLOT

            @pl.when(jn < N_CHUNKS_PER_TILE)
            def _():
                pltpu.make_async_copy(rows[b], acc.at[dstv.at[j]],
                                      ssems[b]).wait()
                pltpu.async_copy(u_src.at[srcv.at[jn]], rows[b], gsems[b])
        return 0

    lax.fori_loop(0, N_CHUNKS_PER_TILE // NSLOT, body, 0)
    for b in range(NSLOT):
        j = N_CHUNKS_PER_TILE - NSLOT + b
        pltpu.make_async_copy(rows[b], acc.at[dstv.at[j]], ssems[b]).wait()
    plsc.subcore_barrier()
    pltpu.sync_copy(acc.at[pl.ds(s * ROWS_PER_TILE, ROWS_PER_TILE)],
                    out_hbm.at[c, pl.ds(s * ROWS_PER_TILE, ROWS_PER_TILE)])


def _scat0_body(src_hbm, dst_hbm, u_hbm, zeros_hbm, out_hbm,
                srcv, dstv, *rest):
    rows, acc, gsems, ssems = rest[:NSLOT], rest[NSLOT], \
        rest[NSLOT + 1:2 * NSLOT + 1], rest[2 * NSLOT + 1:]
    c = lax.axis_index("c")
    s = lax.axis_index("s")
    wid = c * NS + s
    _edge_phase(src_hbm, dst_hbm, u_hbm, zeros_hbm, out_hbm,
                srcv, dstv, rows, acc, gsems, ssems, s, c, wid)


def _scatc_body(pp_hbm, up_hbm, c_hbm, g_hbm, src_hbm, dst_hbm, zeros_hbm,
                out_hbm, u_out, srcv, dstv, *rest):
    rows, acc = rest[:NSLOT], rest[NSLOT]
    p0v, p1v, uv, cv, gv = rest[NSLOT + 1:NSLOT + 6]
    gsems = rest[NSLOT + 6:2 * NSLOT + 6]
    ssems = rest[2 * NSLOT + 6:]
    c = lax.axis_index("c")
    s = lax.axis_index("s")
    wid = c * NS + s
    for quarter in range(8):
        r0 = s * ROWS_PER_TILE + quarter * QROWS
        pltpu.sync_copy(pp_hbm.at[0, pl.ds(r0, QROWS)], p0v)
        pltpu.sync_copy(pp_hbm.at[1, pl.ds(r0, QROWS)], p1v)
        pltpu.sync_copy(up_hbm.at[pl.ds(r0, QROWS)], uv)
        pltpu.sync_copy(c_hbm.at[pl.ds(r0, QROWS)], cv)
        pltpu.sync_copy(g_hbm.at[pl.ds(r0, QROWS)], gv)

        def cstep(i, _):
            for k in range(DP // 16):
                ix = pl.ds(k * 16, 16)
                agg = p0v[i, ix] + p1v[i, ix] + uv[i, ix]
                uv[i, ix] = (1.0 - ALPHA) * cv[i, ix] * agg + gv[i, ix]
            return 0

        lax.fori_loop(0, QROWS, cstep, 0)
        pltpu.sync_copy(uv, u_out.at[pl.ds(r0, QROWS)])
    plsc.subcore_barrier()
    _edge_phase(src_hbm, dst_hbm, u_out, zeros_hbm, out_hbm,
                srcv, dstv, rows, acc, gsems, ssems, s, c, wid)


_EDGE_SCRATCH = lambda: [
    pltpu.VMEM((N_CHUNKS_PER_TILE, E_CHUNK), jnp.int32),
    pltpu.VMEM((N_CHUNKS_PER_TILE, E_CHUNK), jnp.int32),
] + [pltpu.VMEM((E_CHUNK, DP), jnp.float32) for _ in range(NSLOT)] + [
    pltpu.VMEM_SHARED((NP, DP), jnp.float32),
]
_SEMS = lambda: [pltpu.SemaphoreType.DMA for _ in range(2 * NSLOT)]


def _scat0_call(src2d, dst2d, u, zeros48_h):
    return pl.kernel(
        _scat0_body,
        out_type=jax.ShapeDtypeStruct((NC, NP, DP), jnp.float32),
        mesh=_mesh(),
        compiler_params=_SC_PARAMS,
        scratch_types=_EDGE_SCRATCH() + _SEMS(),
    )(src2d, dst2d, u, zeros48_h)


def _scatc_call(parts, u_prev, cb, g, src2d, dst2d, zeros48_h):
    return pl.kernel(
        _scatc_body,
        out_type=(
            jax.ShapeDtypeStruct((NC, NP, DP), jnp.float32),
            jax.ShapeDtypeStruct((NP, DP), jnp.float32),
        ),
        mesh=_mesh(),
        compiler_params=_SC_PARAMS,
        scratch_types=_EDGE_SCRATCH() + [
            pltpu.VMEM((QROWS, DP), jnp.float32),
            pltpu.VMEM((QROWS, DP), jnp.float32),
            pltpu.VMEM((QROWS, DP), jnp.float32),
            pltpu.VMEM((QROWS, DP), jnp.float32),
            pltpu.VMEM((QROWS, DP), jnp.float32),
        ] + _SEMS(),
    )(parts, u_prev, cb, g, src2d, dst2d, zeros48_h)


def kernel(x, edge_index, W1, b1, W2, b2):
    ei = edge_index.astype(jnp.int32)
    pad = jnp.full((2, EP - ei.shape[1]), PAD_NODE, dtype=jnp.int32)
    e = jnp.concatenate([ei, pad], axis=1)
    src2d = e[0].reshape(EP // E_CHUNK, E_CHUNK)
    dst2d = e[1].reshape(EP // E_CHUNK, E_CHUNK)

    x_pad = jnp.pad(x, ((0, NP - N_NODES), (0, 0)))
    w2p = jnp.pad(W2, ((0, 0), (0, DP - N_CLASSES)))
    b2p = jnp.pad(b2, (0, DP - N_CLASSES))

    ones_h = jnp.ones((E_CHUNK, 16), jnp.float32)
    zeros16_h = jnp.zeros((NP, 16), jnp.float32)
    zeros48_h = jnp.zeros((NP, DP), jnp.float32)

    dparts = _deg_call(dst2d, ones_h, zeros16_h)
    p0, p1 = dparts[0], dparts[1]
    u0, cb, g = _mlp_call(x_pad, W1, b1, w2p, b2p, p0, p1)

    parts = _scat0_call(src2d, dst2d, u0, zeros48_h)
    u = u0
    for _ in range(K_STEPS - 1):
        parts, u = _scatc_call(parts, u, cb, g, src2d, dst2d, zeros48_h)

    return _out_call(parts, u, cb, g, p0, p1)


# speedup vs baseline: 1.2468x; 1.2468x over previous
import jax
import jax.numpy as jnp
from jax import lax
from jax.experimental import pallas as pl
from jax.experimental.pallas import tpu as pltpu
from jax.experimental.pallas import tpu_sc as plsc

N_NODES = 10000
N_CLASSES = 40
K_STEPS = 10
ALPHA = 0.1

NC, NS = 2, 16
NW = NC * NS
NP = 10240
PAD_NODE = N_NODES
DP = 48
EP = 327680
E_CHUNK = 128
N_CHUNKS_PER_TILE = EP // NW // E_CHUNK
ROWS_PER_TILE = NP // NS
QROWS = ROWS_PER_TILE // 4

_mesh = lambda: plsc.VectorSubcoreMesh(
    core_axis_name="c", subcore_axis_name="s", num_cores=NC, num_subcores=NS)
_SC_PARAMS = pltpu.CompilerParams(use_tc_tiling_on_sc=False)


def _mlp_body(x_ref, w1_ref, b1_ref, w2_ref, b2_ref, p0_ref, p1_ref,
              u0_ref, c_ref, g_ref):
    x = x_ref[...]
    h = jax.nn.relu(jnp.dot(x, w1_ref[...], preferred_element_type=jnp.float32)
                    + b1_ref[...][None, :])
    h = jnp.dot(h, w2_ref[...], preferred_element_type=jnp.float32) \
        + b2_ref[...][None, :]
    deg = p0_ref[...][:, 0:1] + p1_ref[...][:, 0:1] + 1.0
    dinv = lax.rsqrt(deg)
    row = lax.broadcasted_iota(jnp.int32, (NP, 1), 0)
    live = row < N_NODES
    u0 = jnp.where(live, dinv * h, 0.0)
    u0_ref[...] = u0
    live16 = lax.broadcasted_iota(jnp.int32, (NP, 16), 0) < N_NODES
    c_ref[...] = jnp.where(live16, jnp.ones((NP, 16), jnp.float32) / deg, 0.0)
    g_ref[...] = ALPHA * u0


def _mlp_call(x_pad, w1, b1, w2p, b2p, p0, p1):
    return pl.pallas_call(
        _mlp_body,
        out_shape=(
            jax.ShapeDtypeStruct((NP, DP), jnp.float32),
            jax.ShapeDtypeStruct((NP, 16), jnp.float32),
            jax.ShapeDtypeStruct((NP, DP), jnp.float32),
        ),
    )(x_pad, w1, b1, w2p, b2p, p0, p1)


def _out_body(pp_ref, u_ref, c_ref, g_ref, p0_ref, p1_ref, o_ref):
    u10 = (1.0 - ALPHA) * c_ref[...][:, 0:1] * (
        pp_ref[0] + pp_ref[1] + u_ref[...]) + g_ref[...]
    deg = p0_ref[...][:N_NODES, 0:1] + p1_ref[...][:N_NODES, 0:1] + 1.0
    z = jnp.sqrt(deg) * u10[:N_NODES, :N_CLASSES]
    m = jnp.max(z, axis=1, keepdims=True)
    e = jnp.exp(z - m)
    o_ref[...] = (z - m) - jnp.log(jnp.sum(e, axis=1, keepdims=True))


def _out_call(parts, u, cb, g, p0, p1):
    return pl.pallas_call(
        _out_body,
        out_shape=jax.ShapeDtypeStruct((N_NODES, N_CLASSES), jnp.float32),
    )(parts, u, cb, g, p0, p1)


def _deg_body(dst_hbm, ones_hbm, zeros_hbm, out_hbm, dstv, onesv, acc, sem):
    c = lax.axis_index("c")
    s = lax.axis_index("s")
    wid = c * NS + s
    pltpu.sync_copy(zeros_hbm.at[pl.ds(s * ROWS_PER_TILE, ROWS_PER_TILE)],
                    acc.at[pl.ds(s * ROWS_PER_TILE, ROWS_PER_TILE)])
    pltpu.sync_copy(ones_hbm, onesv)
    pltpu.sync_copy(dst_hbm.at[pl.ds(wid * N_CHUNKS_PER_TILE, N_CHUNKS_PER_TILE)],
                    dstv)
    plsc.subcore_barrier()

    def step(j, _):
        pltpu.sync_copy(onesv, acc.at[dstv.at[j]], add=True)
        return 0

    lax.fori_loop(0, N_CHUNKS_PER_TILE, step, 0)
    plsc.subcore_barrier()
    pltpu.sync_copy(acc.at[pl.ds(s * ROWS_PER_TILE, ROWS_PER_TILE)],
                    out_hbm.at[c, pl.ds(s * ROWS_PER_TILE, ROWS_PER_TILE)])


def _deg_call(dst2d, ones_h, zeros16_h):
    return pl.kernel(
        _deg_body,
        out_type=jax.ShapeDtypeStruct((NC, NP, 16), jnp.float32),
        mesh=_mesh(),
        compiler_params=_SC_PARAMS,
        scratch_types=[
            pltpu.VMEM((N_CHUNKS_PER_TILE, E_CHUNK), jnp.int32),
            pltpu.VMEM((E_CHUNK, 16), jnp.float32),
            pltpu.VMEM_SHARED((NP, 16), jnp.float32),
            pltpu.SemaphoreType.DMA,
        ],
    )(dst2d, ones_h, zeros16_h)


def _edge_prep(src_hbm, dst_hbm, zeros_hbm, srcv, dstv, acc, semp, s, wid):
    d0 = pltpu.async_copy(
        zeros_hbm.at[pl.ds(s * ROWS_PER_TILE, ROWS_PER_TILE)],
        acc.at[pl.ds(s * ROWS_PER_TILE, ROWS_PER_TILE)], semp)
    d1 = pltpu.async_copy(
        src_hbm.at[pl.ds(wid * N_CHUNKS_PER_TILE, N_CHUNKS_PER_TILE)],
        srcv, semp)
    d2 = pltpu.async_copy(
        dst_hbm.at[pl.ds(wid * N_CHUNKS_PER_TILE, N_CHUNKS_PER_TILE)],
        dstv, semp)
    return (d0, d1, d2)


def _edge_run(u_src, out_hbm, srcv, dstv, rows0, rows1, acc,
              sem0, sem1, prep, s, c):
    for d in prep:
        d.wait()
    plsc.subcore_barrier()

    pltpu.make_async_copy(u_src.at[srcv.at[0]], rows0, sem0).start()

    def step(jj, _):
        j0 = jj * 2
        j1 = j0 + 1
        pltpu.make_async_copy(u_src.at[srcv.at[j1]], rows1, sem1).start()
        pltpu.make_async_copy(u_src.at[srcv.at[j0]], rows0, sem0).wait()
        pltpu.sync_copy(rows0, acc.at[dstv.at[j0]], add=True)
        j2 = lax.rem(j0 + 2, N_CHUNKS_PER_TILE)
        pltpu.make_async_copy(u_src.at[srcv.at[j2]], rows0, sem0).start()
        pltpu.make_async_copy(u_src.at[srcv.at[j1]], rows1, sem1).wait()
        pltpu.sync_copy(rows1, acc.at[dstv.at[j1]], add=True)
        return 0

    lax.fori_loop(0, N_CHUNKS_PER_TILE // 2, step, 0)
    pltpu.make_async_copy(u_src.at[srcv.at[0]], rows0, sem0).wait()
    plsc.subcore_barrier()
    pltpu.sync_copy(acc.at[pl.ds(s * ROWS_PER_TILE, ROWS_PER_TILE)],
                    out_hbm.at[c, pl.ds(s * ROWS_PER_TILE, ROWS_PER_TILE)])


def _scat0_body(src_hbm, dst_hbm, u_hbm, zeros_hbm, out_hbm,
                srcv, dstv, rows0, rows1, acc, sem0, sem1, semp):
    c = lax.axis_index("c")
    s = lax.axis_index("s")
    wid = c * NS + s
    prep = _edge_prep(src_hbm, dst_hbm, zeros_hbm, srcv, dstv, acc, semp,
                      s, wid)
    _edge_run(u_hbm, out_hbm, srcv, dstv, rows0, rows1, acc,
              sem0, sem1, prep, s, c)


def _scatc_body(pp_hbm, up_hbm, c_hbm, g_hbm, src_hbm, dst_hbm, zeros_hbm,
                out_hbm, u_out, srcv, dstv, rows0, rows1, acc,
                p0v, p1v, uv, cv, gv, sem0, sem1, semp, semc):
    c = lax.axis_index("c")
    s = lax.axis_index("s")
    wid = c * NS + s
    prep = _edge_prep(src_hbm, dst_hbm, zeros_hbm, srcv, dstv, acc, semp,
                      s, wid)
    store_d = None
    for quarter in range(4):
        r0 = s * ROWS_PER_TILE + quarter * QROWS
        loads = [
            pltpu.async_copy(pp_hbm.at[0, pl.ds(r0, QROWS)], p0v, semc),
            pltpu.async_copy(pp_hbm.at[1, pl.ds(r0, QROWS)], p1v, semc),
            pltpu.async_copy(c_hbm.at[pl.ds(r0, QROWS)], cv, semc),
            pltpu.async_copy(g_hbm.at[pl.ds(r0, QROWS)], gv, semc),
        ]
        if store_d is not None:
            store_d.wait()
        loads.append(pltpu.async_copy(up_hbm.at[pl.ds(r0, QROWS)], uv, semc))
        for d in loads:
            d.wait()

        def cstep(i, _):
            crow = cv[i, :]
            for k in range(DP // 16):
                ix = pl.ds(k * 16, 16)
                agg = p0v[i, ix] + p1v[i, ix] + uv[i, ix]
                uv[i, ix] = (1.0 - ALPHA) * crow * agg + gv[i, ix]
            return 0

        lax.fori_loop(0, QROWS, cstep, 0)
        store_d = pltpu.async_copy(uv, u_out.at[pl.ds(r0, QROWS)], semc)
    store_d.wait()
    plsc.subcore_barrier()
    _edge_run(u_out, out_hbm, srcv, dstv, rows0, rows1, acc,
              sem0, sem1, prep, s, c)


_EDGE_SCRATCH = lambda: [
    pltpu.VMEM((N_CHUNKS_PER_TILE, E_CHUNK), jnp.int32),
    pltpu.VMEM((N_CHUNKS_PER_TILE, E_CHUNK), jnp.int32),
    pltpu.VMEM((E_CHUNK, DP), jnp.float32),
    pltpu.VMEM((E_CHUNK, DP), jnp.float32),
    pltpu.VMEM_SHARED((NP, DP), jnp.float32),
]


def _scat0_call(src2d, dst2d, u, zeros48_h):
    return pl.kernel(
        _scat0_body,
        out_type=jax.ShapeDtypeStruct((NC, NP, DP), jnp.float32),
        mesh=_mesh(),
        compiler_params=_SC_PARAMS,
        scratch_types=_EDGE_SCRATCH() + [pltpu.SemaphoreType.DMA] * 3,
    )(src2d, dst2d, u, zeros48_h)


def _scatc_call(parts, u_prev, cb, g, src2d, dst2d, zeros48_h):
    return pl.kernel(
        _scatc_body,
        out_type=(
            jax.ShapeDtypeStruct((NC, NP, DP), jnp.float32),
            jax.ShapeDtypeStruct((NP, DP), jnp.float32),
        ),
        mesh=_mesh(),
        compiler_params=_SC_PARAMS,
        scratch_types=_EDGE_SCRATCH() + [
            pltpu.VMEM((QROWS, DP), jnp.float32),
            pltpu.VMEM((QROWS, DP), jnp.float32),
            pltpu.VMEM((QROWS, DP), jnp.float32),
            pltpu.VMEM((QROWS, 16), jnp.float32),
            pltpu.VMEM((QROWS, DP), jnp.float32),
        ] + [pltpu.SemaphoreType.DMA] * 4,
    )(parts, u_prev, cb, g, src2d, dst2d, zeros48_h)


def kernel(x, edge_index, W1, b1, W2, b2):
    ei = edge_index.astype(jnp.int32)
    pad = jnp.full((2, EP - ei.shape[1]), PAD_NODE, dtype=jnp.int32)
    e = jnp.concatenate([ei, pad], axis=1)
    src2d = e[0].reshape(EP // E_CHUNK, E_CHUNK)
    dst2d = e[1].reshape(EP // E_CHUNK, E_CHUNK)

    x_pad = jnp.pad(x, ((0, NP - N_NODES), (0, 0)))
    w2p = jnp.pad(W2, ((0, 0), (0, DP - N_CLASSES)))
    b2p = jnp.pad(b2, (0, DP - N_CLASSES))

    ones_h = jnp.ones((E_CHUNK, 16), jnp.float32)
    zeros16_h = jnp.zeros((NP, 16), jnp.float32)
    zeros48_h = jnp.zeros((NP, DP), jnp.float32)

    dparts = _deg_call(dst2d, ones_h, zeros16_h)
    p0, p1 = dparts[0], dparts[1]
    u0, cb, g = _mlp_call(x_pad, W1, b1, w2p, b2p, p0, p1)

    parts = _scat0_call(src2d, dst2d, u0, zeros48_h)
    u = u0
    for _ in range(K_STEPS - 1):
        parts, u = _scatc_call(parts, u, cb, g, src2d, dst2d, zeros48_h)

    return _out_call(parts, u, cb, g, p0, p1)


# speedup vs baseline: 1.2493x; 1.0020x over previous
import jax
import jax.numpy as jnp
from jax import lax
from jax.experimental import pallas as pl
from jax.experimental.pallas import tpu as pltpu
from jax.experimental.pallas import tpu_sc as plsc

N_NODES = 10000
N_CLASSES = 40
K_STEPS = 10
ALPHA = 0.1

NC, NS = 2, 16
NW = NC * NS
NP = 10240
PAD_NODE = N_NODES
DP = 48
EP = 327680
E_CHUNK = 256
N_CHUNKS_PER_TILE = EP // NW // E_CHUNK
ROWS_PER_TILE = NP // NS
QROWS = ROWS_PER_TILE // 4

_mesh = lambda: plsc.VectorSubcoreMesh(
    core_axis_name="c", subcore_axis_name="s", num_cores=NC, num_subcores=NS)
_SC_PARAMS = pltpu.CompilerParams(use_tc_tiling_on_sc=False)


def _mlp_body(x_ref, w1_ref, b1_ref, w2_ref, b2_ref, p0_ref, p1_ref,
              u0_ref, c_ref, g_ref):
    x = x_ref[...]
    h = jax.nn.relu(jnp.dot(x, w1_ref[...], preferred_element_type=jnp.float32)
                    + b1_ref[...][None, :])
    h = jnp.dot(h, w2_ref[...], preferred_element_type=jnp.float32) \
        + b2_ref[...][None, :]
    deg = p0_ref[...][:, 0:1] + p1_ref[...][:, 0:1] + 1.0
    dinv = lax.rsqrt(deg)
    row = lax.broadcasted_iota(jnp.int32, (NP, 1), 0)
    live = row < N_NODES
    u0 = jnp.where(live, dinv * h, 0.0)
    u0_ref[...] = u0
    live16 = lax.broadcasted_iota(jnp.int32, (NP, 16), 0) < N_NODES
    c_ref[...] = jnp.where(live16, jnp.ones((NP, 16), jnp.float32) / deg, 0.0)
    g_ref[...] = ALPHA * u0


def _mlp_call(x_pad, w1, b1, w2p, b2p, p0, p1):
    return pl.pallas_call(
        _mlp_body,
        out_shape=(
            jax.ShapeDtypeStruct((NP, DP), jnp.float32),
            jax.ShapeDtypeStruct((NP, 16), jnp.float32),
            jax.ShapeDtypeStruct((NP, DP), jnp.float32),
        ),
    )(x_pad, w1, b1, w2p, b2p, p0, p1)


def _out_body(pp_ref, u_ref, c_ref, g_ref, p0_ref, p1_ref, o_ref):
    u10 = (1.0 - ALPHA) * c_ref[...][:, 0:1] * (
        pp_ref[0] + pp_ref[1] + u_ref[...]) + g_ref[...]
    deg = p0_ref[...][:N_NODES, 0:1] + p1_ref[...][:N_NODES, 0:1] + 1.0
    z = jnp.sqrt(deg) * u10[:N_NODES, :N_CLASSES]
    m = jnp.max(z, axis=1, keepdims=True)
    e = jnp.exp(z - m)
    o_ref[...] = (z - m) - jnp.log(jnp.sum(e, axis=1, keepdims=True))


def _out_call(parts, u, cb, g, p0, p1):
    return pl.pallas_call(
        _out_body,
        out_shape=jax.ShapeDtypeStruct((N_NODES, N_CLASSES), jnp.float32),
    )(parts, u, cb, g, p0, p1)


def _deg_body(dst_hbm, ones_hbm, zeros_hbm, out_hbm, dstv, onesv, acc, sem):
    c = lax.axis_index("c")
    s = lax.axis_index("s")
    wid = c * NS + s
    pltpu.sync_copy(zeros_hbm.at[pl.ds(s * ROWS_PER_TILE, ROWS_PER_TILE)],
                    acc.at[pl.ds(s * ROWS_PER_TILE, ROWS_PER_TILE)])
    pltpu.sync_copy(ones_hbm, onesv)
    pltpu.sync_copy(dst_hbm.at[pl.ds(wid * N_CHUNKS_PER_TILE, N_CHUNKS_PER_TILE)],
                    dstv)
    plsc.subcore_barrier()

    def step(j, _):
        pltpu.sync_copy(onesv, acc.at[dstv.at[j]], add=True)
        return 0

    lax.fori_loop(0, N_CHUNKS_PER_TILE, step, 0)
    plsc.subcore_barrier()
    pltpu.sync_copy(acc.at[pl.ds(s * ROWS_PER_TILE, ROWS_PER_TILE)],
                    out_hbm.at[c, pl.ds(s * ROWS_PER_TILE, ROWS_PER_TILE)])


def _deg_call(dst2d, ones_h, zeros16_h):
    return pl.kernel(
        _deg_body,
        out_type=jax.ShapeDtypeStruct((NC, NP, 16), jnp.float32),
        mesh=_mesh(),
        compiler_params=_SC_PARAMS,
        scratch_types=[
            pltpu.VMEM((N_CHUNKS_PER_TILE, E_CHUNK), jnp.int32),
            pltpu.VMEM((E_CHUNK, 16), jnp.float32),
            pltpu.VMEM_SHARED((NP, 16), jnp.float32),
            pltpu.SemaphoreType.DMA,
        ],
    )(dst2d, ones_h, zeros16_h)


def _edge_prep(src_hbm, dst_hbm, zeros_hbm, srcv, dstv, acc, semp, s, wid):
    d0 = pltpu.async_copy(
        zeros_hbm.at[pl.ds(s * ROWS_PER_TILE, ROWS_PER_TILE)],
        acc.at[pl.ds(s * ROWS_PER_TILE, ROWS_PER_TILE)], semp)
    d1 = pltpu.async_copy(
        src_hbm.at[pl.ds(wid * N_CHUNKS_PER_TILE, N_CHUNKS_PER_TILE)],
        srcv, semp)
    d2 = pltpu.async_copy(
        dst_hbm.at[pl.ds(wid * N_CHUNKS_PER_TILE, N_CHUNKS_PER_TILE)],
        dstv, semp)
    return (d0, d1, d2)


def _edge_run(u_src, out_hbm, srcv, dstv, rows0, rows1, acc,
              sem0, sem1, prep, s, c):
    for d in prep:
        d.wait()
    plsc.subcore_barrier()

    pltpu.make_async_copy(u_src.at[srcv.at[0]], rows0, sem0).start()

    def step(jj, _):
        j0 = jj * 2
        j1 = j0 + 1
        pltpu.make_async_copy(u_src.at[srcv.at[j1]], rows1, sem1).start()
        pltpu.make_async_copy(u_src.at[srcv.at[j0]], rows0, sem0).wait()
        pltpu.sync_copy(rows0, acc.at[dstv.at[j0]], add=True)
        j2 = lax.rem(j0 + 2, N_CHUNKS_PER_TILE)
        pltpu.make_async_copy(u_src.at[srcv.at[j2]], rows0, sem0).start()
        pltpu.make_async_copy(u_src.at[srcv.at[j1]], rows1, sem1).wait()
        pltpu.sync_copy(rows1, acc.at[dstv.at[j1]], add=True)
        return 0

    lax.fori_loop(0, N_CHUNKS_PER_TILE // 2, step, 0)
    pltpu.make_async_copy(u_src.at[srcv.at[0]], rows0, sem0).wait()
    plsc.subcore_barrier()
    pltpu.sync_copy(acc.at[pl.ds(s * ROWS_PER_TILE, ROWS_PER_TILE)],
                    out_hbm.at[c, pl.ds(s * ROWS_PER_TILE, ROWS_PER_TILE)])


def _scat0_body(src_hbm, dst_hbm, u_hbm, zeros_hbm, out_hbm,
                srcv, dstv, rows0, rows1, acc, sem0, sem1, semp):
    c = lax.axis_index("c")
    s = lax.axis_index("s")
    wid = c * NS + s
    prep = _edge_prep(src_hbm, dst_hbm, zeros_hbm, srcv, dstv, acc, semp,
                      s, wid)
    _edge_run(u_hbm, out_hbm, srcv, dstv, rows0, rows1, acc,
              sem0, sem1, prep, s, c)


def _scatc_body(pp_hbm, up_hbm, c_hbm, g_hbm, src_hbm, dst_hbm, zeros_hbm,
                out_hbm, u_out, srcv, dstv, rows0, rows1, acc,
                p0v, p1v, uv, cv, gv, sem0, sem1, semp, semc):
    c = lax.axis_index("c")
    s = lax.axis_index("s")
    wid = c * NS + s
    prep = _edge_prep(src_hbm, dst_hbm, zeros_hbm, srcv, dstv, acc, semp,
                      s, wid)
    store_d = None
    for quarter in range(4):
        r0 = s * ROWS_PER_TILE + quarter * QROWS
        loads = [
            pltpu.async_copy(pp_hbm.at[0, pl.ds(r0, QROWS)], p0v, semc),
            pltpu.async_copy(pp_hbm.at[1, pl.ds(r0, QROWS)], p1v, semc),
            pltpu.async_copy(c_hbm.at[pl.ds(r0, QROWS)], cv, semc),
            pltpu.async_copy(g_hbm.at[pl.ds(r0, QROWS)], gv, semc),
        ]
        if store_d is not None:
            store_d.wait()
        loads.append(pltpu.async_copy(up_hbm.at[pl.ds(r0, QROWS)], uv, semc))
        for d in loads:
            d.wait()

        def cstep(i, _):
            crow = cv[i, :]
            for k in range(DP // 16):
                ix = pl.ds(k * 16, 16)
                agg = p0v[i, ix] + p1v[i, ix] + uv[i, ix]
                uv[i, ix] = (1.0 - ALPHA) * crow * agg + gv[i, ix]
            return 0

        lax.fori_loop(0, QROWS, cstep, 0)
        store_d = pltpu.async_copy(uv, u_out.at[pl.ds(r0, QROWS)], semc)
    store_d.wait()
    plsc.subcore_barrier()
    _edge_run(u_out, out_hbm, srcv, dstv, rows0, rows1, acc,
              sem0, sem1, prep, s, c)


_EDGE_SCRATCH = lambda: [
    pltpu.VMEM((N_CHUNKS_PER_TILE, E_CHUNK), jnp.int32),
    pltpu.VMEM((N_CHUNKS_PER_TILE, E_CHUNK), jnp.int32),
    pltpu.VMEM((E_CHUNK, DP), jnp.float32),
    pltpu.VMEM((E_CHUNK, DP), jnp.float32),
    pltpu.VMEM_SHARED((NP, DP), jnp.float32),
]


def _scat0_call(src2d, dst2d, u, zeros48_h):
    return pl.kernel(
        _scat0_body,
        out_type=jax.ShapeDtypeStruct((NC, NP, DP), jnp.float32),
        mesh=_mesh(),
        compiler_params=_SC_PARAMS,
        scratch_types=_EDGE_SCRATCH() + [pltpu.SemaphoreType.DMA] * 3,
    )(src2d, dst2d, u, zeros48_h)


def _scatc_call(parts, u_prev, cb, g, src2d, dst2d, zeros48_h):
    return pl.kernel(
        _scatc_body,
        out_type=(
            jax.ShapeDtypeStruct((NC, NP, DP), jnp.float32),
            jax.ShapeDtypeStruct((NP, DP), jnp.float32),
        ),
        mesh=_mesh(),
        compiler_params=_SC_PARAMS,
        scratch_types=_EDGE_SCRATCH() + [
            pltpu.VMEM((QROWS, DP), jnp.float32),
            pltpu.VMEM((QROWS, DP), jnp.float32),
            pltpu.VMEM((QROWS, DP), jnp.float32),
            pltpu.VMEM((QROWS, 16), jnp.float32),
            pltpu.VMEM((QROWS, DP), jnp.float32),
        ] + [pltpu.SemaphoreType.DMA] * 4,
    )(parts, u_prev, cb, g, src2d, dst2d, zeros48_h)


def kernel(x, edge_index, W1, b1, W2, b2):
    ei = edge_index.astype(jnp.int32)
    pad = jnp.full((2, EP - ei.shape[1]), PAD_NODE, dtype=jnp.int32)
    e = jnp.concatenate([ei, pad], axis=1)
    src2d = e[0].reshape(EP // E_CHUNK, E_CHUNK)
    dst2d = e[1].reshape(EP // E_CHUNK, E_CHUNK)

    x_pad = jnp.pad(x, ((0, NP - N_NODES), (0, 0)))
    w2p = jnp.pad(W2, ((0, 0), (0, DP - N_CLASSES)))
    b2p = jnp.pad(b2, (0, DP - N_CLASSES))

    ones_h = jnp.ones((E_CHUNK, 16), jnp.float32)
    zeros16_h = jnp.zeros((NP, 16), jnp.float32)
    zeros48_h = jnp.zeros((NP, DP), jnp.float32)

    dparts = _deg_call(dst2d, ones_h, zeros16_h)
    p0, p1 = dparts[0], dparts[1]
    u0, cb, g = _mlp_call(x_pad, W1, b1, w2p, b2p, p0, p1)

    parts = _scat0_call(src2d, dst2d, u0, zeros48_h)
    u = u0
    for _ in range(K_STEPS - 1):
        parts, u = _scatc_call(parts, u, cb, g, src2d, dst2d, zeros48_h)

    return _out_call(parts, u, cb, g, p0, p1)
